# baseline (speedup 1.0000x reference)
import jax
import jax.numpy as jnp
from jax import lax
from jax.experimental import pallas as pl
from jax.experimental.pallas import tpu as pltpu
from jax.experimental.pallas import tpu_sc as plsc

B, H, S, D = 16, 8, 2048, 128
ROWS = B * H
NTILE = 32
PRE = 32
ROWS_PER_TILE = PRE // NTILE
CH = 512
NCH = S // CH
BR = 4


def _sc_body(src, dst, bufA, bufB, sems):
    c = lax.axis_index("c")
    s = lax.axis_index("s")
    base = (c * 16 + s) * ROWS_PER_TILE
    bufs = (bufA, bufB)

    def sl(i):
        row = base + i // NCH
        off = (i % NCH) * CH
        return (row, pl.ds(off, CH), slice(None))

    def in_copy(i, b):
        return pltpu.make_async_copy(src.at[sl(i)], bufs[b], sems.at[b])

    def out_copy(i, b):
        return pltpu.make_async_copy(bufs[b], dst.at[sl(i)], sems.at[2 + b])

    n = ROWS_PER_TILE * NCH
    in_copy(0, 0).start()
    for i in range(n):
        b = i % 2
        nb = 1 - b
        in_copy(i, b).wait()
        if i + 1 < n:
            if i >= 1:
                out_copy(i - 1, nb).wait()
            in_copy(i + 1, nb).start()
        out_copy(i, b).start()
    out_copy(n - 2, n % 2).wait()
    out_copy(n - 1, (n - 1) % 2).wait()


def _tc_copy_body(x_ref, o_ref):
    o_ref[...] = x_ref[...]


def _tc_finish_body(alias_ref, x_ref, o_ref):
    o_ref[...] = x_ref[...]


def kernel(k_val, v_val, k_cache, v_cache):
    k2 = k_val.reshape(ROWS, S, D)
    v2 = v_val.reshape(ROWS, S, D)

    sc_fn = pl.kernel(
        _sc_body,
        out_type=jax.ShapeDtypeStruct((ROWS, S, D), jnp.float32),
        mesh=plsc.VectorSubcoreMesh(core_axis_name="c", subcore_axis_name="s"),
        scratch_types=[
            pltpu.MemorySpace.VMEM((CH, D), jnp.float32),
            pltpu.MemorySpace.VMEM((CH, D), jnp.float32),
            pltpu.SemaphoreType.DMA((4,)),
        ],
    )
    vo_pre = sc_fn(v2)

    spec = pl.BlockSpec((BR, S, D), lambda i: (i, 0, 0))
    ko = pl.pallas_call(
        _tc_copy_body,
        grid=(ROWS // BR,),
        in_specs=[spec],
        out_specs=spec,
        out_shape=jax.ShapeDtypeStruct((ROWS, S, D), jnp.float32),
    )(k2)

    suf_spec = pl.BlockSpec((BR, S, D), lambda i: (PRE // BR + i, 0, 0))
    vo = pl.pallas_call(
        _tc_finish_body,
        grid=((ROWS - PRE) // BR,),
        in_specs=[pl.BlockSpec(memory_space=pl.ANY), suf_spec],
        out_specs=suf_spec,
        out_shape=jax.ShapeDtypeStruct((ROWS, S, D), jnp.float32),
        input_output_aliases={0: 0},
    )(vo_pre, v2)

    return ko.reshape(B, H, S, D), vo.reshape(B, H, S, D)


# speedup vs baseline: 1.0001x; 1.0001x over previous
import jax
import jax.numpy as jnp
from jax import lax
from jax.experimental import pallas as pl
from jax.experimental.pallas import tpu as pltpu
from jax.experimental.pallas import tpu_sc as plsc

B, H, S, D = 16, 8, 2048, 128
ROWS = B * H
NTILE = 32
PRE = 32
ROWS_PER_TILE = PRE // NTILE
CH = 512
NCH = S // CH
BR = 4


def _sc_body(src, dst, bufA, bufB, sems):
    c = lax.axis_index("c")
    s = lax.axis_index("s")
    base = (c * 16 + s) * ROWS_PER_TILE
    bufs = (bufA, bufB)

    def sl(i):
        row = base + i // NCH
        off = (i % NCH) * CH
        return (row, pl.ds(off, CH), slice(None))

    def in_copy(i, b):
        return pltpu.make_async_copy(src.at[sl(i)], bufs[b], sems.at[b])

    def out_copy(i, b):
        return pltpu.make_async_copy(bufs[b], dst.at[sl(i)], sems.at[2 + b])

    n = ROWS_PER_TILE * NCH
    in_copy(0, 0).start()
    for i in range(n):
        b = i % 2
        nb = 1 - b
        in_copy(i, b).wait()
        if i + 1 < n:
            if i >= 1:
                out_copy(i - 1, nb).wait()
            in_copy(i + 1, nb).start()
        out_copy(i, b).start()
    out_copy(n - 2, n % 2).wait()
    out_copy(n - 1, (n - 1) % 2).wait()


def _tc_copy_body(x_ref, o_ref):
    o_ref[...] = x_ref[...]


def _tc_finish_body(alias_ref, x_ref, o_ref):
    o_ref[...] = x_ref[...]


def kernel(k_val, v_val, k_cache, v_cache):
    k2 = k_val.reshape(ROWS, S, D)
    v2 = v_val.reshape(ROWS, S, D)

    spec = pl.BlockSpec((BR, S, D), lambda i: (i, 0, 0))
    ko = pl.pallas_call(
        _tc_copy_body,
        grid=(ROWS // BR,),
        in_specs=[spec],
        out_specs=spec,
        out_shape=jax.ShapeDtypeStruct((ROWS, S, D), jnp.float32),
    )(k2)

    sc_fn = pl.kernel(
        _sc_body,
        out_type=jax.ShapeDtypeStruct((ROWS, S, D), jnp.float32),
        mesh=plsc.VectorSubcoreMesh(core_axis_name="c", subcore_axis_name="s"),
        scratch_types=[
            pltpu.MemorySpace.VMEM((CH, D), jnp.float32),
            pltpu.MemorySpace.VMEM((CH, D), jnp.float32),
            pltpu.SemaphoreType.DMA((4,)),
        ],
    )
    vo_pre = sc_fn(v2)

    suf_spec = pl.BlockSpec((BR, S, D), lambda i: (PRE // BR + i, 0, 0))
    vo = pl.pallas_call(
        _tc_finish_body,
        grid=((ROWS - PRE) // BR,),
        in_specs=[pl.BlockSpec(memory_space=pl.ANY), suf_spec],
        out_specs=suf_spec,
        out_shape=jax.ShapeDtypeStruct((ROWS, S, D), jnp.float32),
        input_output_aliases={0: 0},
    )(vo_pre, v2)

    return ko.reshape(B, H, S, D), vo.reshape(B, H, S, D)
